# Initial kernel scaffold; baseline (speedup 1.0000x reference)
import jax
import jax.numpy as jnp
from jax.experimental import pallas as pl


def kernel(x, edge_index, W0, b0, g0, bt0, W1, b1, g1, bt1, W2, b2):
    raise NotImplementedError("write your pallas kernel here")


# speedup vs baseline: 8.2611x; 8.2611x over previous
import functools

import jax
import jax.numpy as jnp
from jax import lax
from jax.experimental import pallas as pl
from jax.experimental.pallas import tpu as pltpu
from jax.experimental.pallas import tpu_sc as plsc

N = 10000
NP = 10240
E = 320000
D_IN = 128
D_H = 256
D_OUT = 128
BN_EPS = 1e-5

NC = 2
NS = 16
STRIPE = NP // NS
CH = 80
EPW = E // NS
NCHUNK = EPW // CH

_MESH = plsc.VectorSubcoreMesh(
    core_axis_name="c", subcore_axis_name="s", num_cores=NC, num_subcores=NS)


def _deg_body(dst_hbm, zeros_hbm, deg_hbm, idx_v, ones_v, acc, sem):
    c = lax.axis_index("c")
    s = lax.axis_index("s")

    @pl.when(c == 0)
    def _():
        pltpu.sync_copy(zeros_hbm.at[pl.ds(s * STRIPE, STRIPE)],
                        acc.at[pl.ds(s * STRIPE, STRIPE)])
        for i in range(CH // 16):
            ones_v[0, pl.ds(i * 16, 16)] = jnp.full((16,), 1.0, jnp.float32)
        plsc.subcore_barrier()

        base = s * EPW

        def step(k, _):
            off = pl.multiple_of(base + k * CH, 8)
            pltpu.sync_copy(dst_hbm.at[pl.ds(off, CH)], idx_v.at[0])
            pltpu.sync_copy(ones_v.at[0], acc.at[idx_v.at[0]], add=True)
            return _

        lax.fori_loop(0, NCHUNK, step, 0)
        plsc.subcore_barrier()
        pltpu.sync_copy(acc.at[pl.ds(s * STRIPE, STRIPE)],
                        deg_hbm.at[pl.ds(s * STRIPE, STRIPE)])


def _deg_kernel(dst, zeros1d):
    return pl.kernel(
        _deg_body,
        out_type=jax.ShapeDtypeStruct((NP,), jnp.float32),
        mesh=_MESH,
        scratch_types=[
            pltpu.VMEM((1, CH), jnp.int32),
            pltpu.VMEM((1, CH), jnp.float32),
            pltpu.VMEM_SHARED((NP,), jnp.float32),
            pltpu.SemaphoreType.DMA,
        ],
    )(dst, zeros1d)


def _agg_body(src_hbm, dst_hbm, hw0_hbm, hw1_hbm, zeros_hbm,
              s0_hbm, s1_hbm, idx_v, rows_v, acc, sem):
    c = lax.axis_index("c")
    s = lax.axis_index("s")

    def run(hw_hbm, out_hbm):
        pltpu.sync_copy(zeros_hbm, acc.at[pl.ds(s * STRIPE, STRIPE)])
        plsc.subcore_barrier()

        base = s * EPW

        def step(k, _):
            off = pl.multiple_of(base + k * CH, 8)
            pltpu.sync_copy(src_hbm.at[pl.ds(off, CH)], idx_v.at[0])
            pltpu.sync_copy(dst_hbm.at[pl.ds(off, CH)], idx_v.at[1])
            pltpu.async_copy(hw_hbm.at[idx_v.at[0]], rows_v, sem).wait()
            pltpu.sync_copy(rows_v, acc.at[idx_v.at[1]], add=True)
            return _

        lax.fori_loop(0, NCHUNK, step, 0)
        plsc.subcore_barrier()
        pltpu.sync_copy(acc.at[pl.ds(s * STRIPE, STRIPE)],
                        out_hbm.at[pl.ds(s * STRIPE, STRIPE)])

    @pl.when(c == 0)
    def _():
        run(hw0_hbm, s0_hbm)

    @pl.when(c == 1)
    def _():
        run(hw1_hbm, s1_hbm)


def _agg_kernel(src, dst, hw0, hw1, zeros, half_w):
    return pl.kernel(
        _agg_body,
        out_type=(jax.ShapeDtypeStruct((NP, half_w), jnp.float32),
                  jax.ShapeDtypeStruct((NP, half_w), jnp.float32)),
        mesh=_MESH,
        scratch_types=[
            pltpu.VMEM((2, CH), jnp.int32),
            pltpu.VMEM((CH, half_w), jnp.float32),
            pltpu.VMEM_SHARED((NP, half_w), jnp.float32),
            pltpu.SemaphoreType.DMA,
        ],
    )(src, dst, hw0, hw1, zeros)


EPW2 = E // (NC * NS)
NCHUNK2 = EPW2 // CH


def _agg2_body(src_hbm, dst_hbm, hw_hbm, zeros_hbm,
               p0_hbm, p1_hbm, idx_v, rows_v, acc, sem):
    c = lax.axis_index("c")
    s = lax.axis_index("s")

    def run(out_hbm, core):
        pltpu.sync_copy(zeros_hbm, acc.at[pl.ds(s * STRIPE, STRIPE)])
        plsc.subcore_barrier()

        base = (core * NS + s) * EPW2

        def step(k, _):
            off = pl.multiple_of(base + k * CH, 8)
            pltpu.sync_copy(src_hbm.at[pl.ds(off, CH)], idx_v.at[0])
            pltpu.sync_copy(dst_hbm.at[pl.ds(off, CH)], idx_v.at[1])
            pltpu.async_copy(hw_hbm.at[idx_v.at[0]], rows_v, sem).wait()
            pltpu.sync_copy(rows_v, acc.at[idx_v.at[1]], add=True)
            return _

        lax.fori_loop(0, NCHUNK2, step, 0)
        plsc.subcore_barrier()
        pltpu.sync_copy(acc.at[pl.ds(s * STRIPE, STRIPE)],
                        out_hbm.at[pl.ds(s * STRIPE, STRIPE)])

    @pl.when(c == 0)
    def _():
        run(p0_hbm, 0)

    @pl.when(c == 1)
    def _():
        run(p1_hbm, 1)


def _agg2_kernel(src, dst, hw, zeros):
    return pl.kernel(
        _agg2_body,
        out_type=(jax.ShapeDtypeStruct((NP, D_OUT), jnp.float32),
                  jax.ShapeDtypeStruct((NP, D_OUT), jnp.float32)),
        mesh=_MESH,
        scratch_types=[
            pltpu.VMEM((2, CH), jnp.int32),
            pltpu.VMEM((CH, D_OUT), jnp.float32),
            pltpu.VMEM_SHARED((NP, D_OUT), jnp.float32),
            pltpu.SemaphoreType.DMA,
        ],
    )(src, dst, hw, zeros)


BR = 1024
GRID = NP // BR


def _tc_first_body(x_ref, w_ref, deg_ref, o0_ref, o1_ref):
    dis = lax.rsqrt(deg_ref[...] + 1.0)
    hw = jnp.dot(x_ref[...], w_ref[...],
                 preferred_element_type=jnp.float32) * dis
    o0_ref[...] = hw[:, :D_H // 2]
    o1_ref[...] = hw[:, D_H // 2:]


def _tc_first(x, w0, deg_col):
    return pl.pallas_call(
        _tc_first_body,
        grid=(GRID,),
        in_specs=[
            pl.BlockSpec((BR, D_IN), lambda r: (r, 0)),
            pl.BlockSpec((D_IN, D_H), lambda r: (0, 0)),
            pl.BlockSpec((BR, 1), lambda r: (r, 0)),
        ],
        out_specs=(pl.BlockSpec((BR, D_H // 2), lambda r: (r, 0)),
                   pl.BlockSpec((BR, D_H // 2), lambda r: (r, 0))),
        out_shape=(jax.ShapeDtypeStruct((NP, D_H // 2), jnp.float32),
                   jax.ShapeDtypeStruct((NP, D_H // 2), jnp.float32)),
    )(x, w0, deg_col)


def _tc_mid_body(s0_ref, s1_ref, h0_ref, h1_ref, deg_ref,
                 b_ref, g_ref, bt_ref, w_ref, *out_refs, split):
    dis = lax.rsqrt(deg_ref[...] + 1.0)
    h = jnp.concatenate(
        [s0_ref[...] + h0_ref[...], s1_ref[...] + h1_ref[...]], axis=1)
    h = h * dis + b_ref[...]
    h = h * (g_ref[...] * (1.0 / jnp.sqrt(1.0 + BN_EPS))) + bt_ref[...]
    h = jnp.maximum(h, 0.0)
    hw = jnp.dot(h, w_ref[...], preferred_element_type=jnp.float32) * dis
    if split:
        half = hw.shape[1] // 2
        out_refs[0][...] = hw[:, :half]
        out_refs[1][...] = hw[:, half:]
    else:
        out_refs[0][...] = hw


def _tc_mid(s0, s1, h0, h1, deg_col, b, g, bt, w, split):
    d_out = w.shape[1]
    half = d_out // 2
    if split:
        out_specs = (pl.BlockSpec((BR, half), lambda r: (r, 0)),
                     pl.BlockSpec((BR, half), lambda r: (r, 0)))
        out_shape = (jax.ShapeDtypeStruct((NP, half), jnp.float32),
                     jax.ShapeDtypeStruct((NP, half), jnp.float32))
    else:
        out_specs = pl.BlockSpec((BR, d_out), lambda r: (r, 0))
        out_shape = jax.ShapeDtypeStruct((NP, d_out), jnp.float32)
    return pl.pallas_call(
        functools.partial(_tc_mid_body, split=split),
        grid=(GRID,),
        in_specs=[
            pl.BlockSpec((BR, D_H // 2), lambda r: (r, 0)),
            pl.BlockSpec((BR, D_H // 2), lambda r: (r, 0)),
            pl.BlockSpec((BR, D_H // 2), lambda r: (r, 0)),
            pl.BlockSpec((BR, D_H // 2), lambda r: (r, 0)),
            pl.BlockSpec((BR, 1), lambda r: (r, 0)),
            pl.BlockSpec((1, D_H), lambda r: (0, 0)),
            pl.BlockSpec((1, D_H), lambda r: (0, 0)),
            pl.BlockSpec((1, D_H), lambda r: (0, 0)),
            pl.BlockSpec((D_H, d_out), lambda r: (0, 0)),
        ],
        out_specs=out_specs,
        out_shape=out_shape,
    )(s0, s1, h0, h1, deg_col, b, g, bt, w)


def _tc_final_body(p0_ref, p1_ref, h_ref, deg_ref, b_ref, o_ref):
    dis = lax.rsqrt(deg_ref[...] + 1.0)
    h = p0_ref[...] + p1_ref[...] + h_ref[...]
    o_ref[...] = h * dis + b_ref[...]


def _tc_final(p0, p1, h, deg_col, b):
    return pl.pallas_call(
        _tc_final_body,
        grid=(GRID,),
        in_specs=[
            pl.BlockSpec((BR, D_OUT), lambda r: (r, 0)),
            pl.BlockSpec((BR, D_OUT), lambda r: (r, 0)),
            pl.BlockSpec((BR, D_OUT), lambda r: (r, 0)),
            pl.BlockSpec((BR, 1), lambda r: (r, 0)),
            pl.BlockSpec((1, D_OUT), lambda r: (0, 0)),
        ],
        out_specs=pl.BlockSpec((BR, D_OUT), lambda r: (r, 0)),
        out_shape=jax.ShapeDtypeStruct((NP, D_OUT), jnp.float32),
    )(p0, p1, h, deg_col, b)


@jax.jit
def kernel(x, edge_index, W0, b0, g0, bt0, W1, b1, g1, bt1, W2, b2):
    src = edge_index[0]
    dst = edge_index[1]

    x_pad = jnp.pad(x, ((0, NP - N), (0, 0)))
    zeros1d = jnp.zeros((NP,), jnp.float32)
    zeros_h = jnp.zeros((STRIPE, D_H // 2), jnp.float32)
    zeros_f = jnp.zeros((STRIPE, D_OUT), jnp.float32)

    deg = _deg_kernel(dst, zeros1d)
    deg_col = deg.reshape(NP, 1)

    h0a, h0b = _tc_first(x_pad, W0, deg_col)
    s0a, s0b = _agg_kernel(src, dst, h0a, h0b, zeros_h, D_H // 2)
    h1a, h1b = _tc_mid(s0a, s0b, h0a, h0b, deg_col,
                       b0.reshape(1, -1), g0.reshape(1, -1),
                       bt0.reshape(1, -1), W1, split=True)
    s1a, s1b = _agg_kernel(src, dst, h1a, h1b, zeros_h, D_H // 2)
    h2 = _tc_mid(s1a, s1b, h1a, h1b, deg_col,
                 b1.reshape(1, -1), g1.reshape(1, -1),
                 bt1.reshape(1, -1), W2, split=False)
    p0, p1 = _agg2_kernel(src, dst, h2, zeros_f)

    out = _tc_final(p0, p1, h2, deg_col, b2.reshape(1, -1))
    return out[:N]
